# baseline (speedup 1.0000x reference)
import jax
import jax.numpy as jnp
from jax import lax
from jax.experimental import pallas as pl
from jax.experimental.pallas import tpu as pltpu
from jax.experimental.pallas import tpu_sc as plsc

VOCAB = 1000000
DIM = 64
B = 4096
S = 200
MAXS = 2048

NC = 2
NS = 16
NW = NC * NS
LANES = 16
BC = 256
NBCH = B // BC
NCHUNK = S * NBCH
CPW = NCHUNK // NW


def _body(ids_hbm, embed_hbm, pos_hbm, out_hbm,
          iv0, iv1, r0, r1, t0, t1, pos_v,
          si0, si1, sg0, sg1, so0, so1):
    iv = [iv0, iv1]
    rows = [r0, r1]
    tr = [t0, t1]
    si = [si0, si1]
    sg = [sg0, sg1]
    so = [so0, so1]

    wid = lax.axis_index("s") * NC + lax.axis_index("c")
    g0 = wid * CPW
    iota = lax.iota(jnp.int32, LANES)
    fhi = [(iota + LANES * fb) >> 3 for fb in range(DIM // LANES)]
    flo = [(iota + LANES * fb) & 7 for fb in range(DIM // LANES)]

    for k in range(2):
        pltpu.sync_copy(pos_hbm.at[:, k], pos_v.at[:, k])

    def schunk(t):
        g = g0 + t
        bc = pl.multiple_of((g & (NBCH - 1)) << 8, BC)
        return g >> 4, bc

    def idesc(t, p):
        g = g0 + t
        s = g >> 4
        s_hi, s_lo, b_hi = s >> 3, s & 7, g & (NBCH - 1)
        b2 = b_hi << 1
        return (
            pltpu.make_async_copy(
                ids_hbm.at[s_hi, b2, s_lo], iv[p].at[pl.ds(0, 128)], si[p]),
            pltpu.make_async_copy(
                ids_hbm.at[s_hi, b2 + 1, s_lo], iv[p].at[pl.ds(128, 128)],
                si[p]),
        )

    def gdesc(t, p):
        return (
            pltpu.make_async_copy(
                embed_hbm.at[iv[p].at[pl.ds(0, 128)]],
                rows[p].at[pl.ds(0, 128)], sg[p]),
            pltpu.make_async_copy(
                embed_hbm.at[iv[p].at[pl.ds(128, 128)]],
                rows[p].at[pl.ds(128, 128)], sg[p]),
        )

    def odesc(t, p):
        s, bc = schunk(t)
        return pltpu.make_async_copy(
            tr[p].at[:, pl.ds(0, BC)], out_hbm.at[s, :, pl.ds(bc, BC)], so[p])

    for x in idesc(0, 0):
        x.start()
    for x in idesc(1, 1):
        x.start()
    for x in idesc(0, 0):
        x.wait()
    for x in gdesc(0, 0):
        x.start()

    def chunk_iter(tt, carry):
        for p in range(2):
            t = 2 * tt + p
            for x in gdesc(t, p):
                x.wait()

            @pl.when(t + 2 < CPW)
            def _():
                for x in idesc(t + 2, p):
                    x.start()

            @pl.when(t + 1 < CPW)
            def _():
                for x in idesc(t + 1, 1 - p):
                    x.wait()
                for x in gdesc(t + 1, 1 - p):
                    x.start()

            @pl.when(t >= 2)
            def _():
                odesc(t - 2, p).wait()

            s, _bc = schunk(t)
            shi_sp = jnp.full((LANES,), s >> 7, jnp.int32)
            slo_sp = jnp.full((LANES,), s & 127, jnp.int32)
            posv = [plsc.load_gather(pos_v, [fhi[fb], shi_sp, flo[fb], slo_sp])
                    for fb in range(DIM // LANES)]

            def jbody(j4, c2, p=p, posv=posv):
                for ju in range(4):
                    j = 4 * j4 + ju
                    jsp = jnp.full((LANES,), j, jnp.int32)
                    for fb in range(DIM // LANES):
                        v = rows[p][j, pl.ds(LANES * fb, LANES)]
                        plsc.store_scatter(
                            tr[p], [iota + LANES * fb, jsp], v + posv[fb])
                return c2

            lax.fori_loop(0, BC // 4, jbody, 0)
            odesc(t, p).start()
        return carry

    lax.fori_loop(0, CPW // 2, chunk_iter, 0)

    odesc(CPW - 2, 0).wait()
    odesc(CPW - 1, 1).wait()


@jax.jit
def _run(ids4, embed, pos4):
    mesh = plsc.VectorSubcoreMesh(core_axis_name="c", subcore_axis_name="s")
    f = pl.kernel(
        _body,
        out_type=jax.ShapeDtypeStruct((S, DIM, B), jnp.float32),
        mesh=mesh,
        scratch_types=[
            pltpu.VMEM((2 * 128,), jnp.int32),
            pltpu.VMEM((2 * 128,), jnp.int32),
            pltpu.VMEM((BC, DIM), jnp.float32),
            pltpu.VMEM((BC, DIM), jnp.float32),
            pltpu.VMEM((DIM, BC + 1), jnp.float32),
            pltpu.VMEM((DIM, BC + 1), jnp.float32),
            pltpu.VMEM((8, 2, 8, 128), jnp.float32),
        ] + [pltpu.SemaphoreType.DMA] * 6,
        compiler_params=pltpu.CompilerParams(
            use_tc_tiling_on_sc=False, needs_layout_passes=False),
    )
    return f(ids4, embed, pos4)


def kernel(input_ids, embed, pos):
    ids4 = (input_ids.astype(jnp.int32).T
            .reshape(S // 8, 8, B // 128, 128).transpose(0, 2, 1, 3))
    pos4 = pos[0].T.reshape(8, 8, MAXS // 128, 128).transpose(0, 2, 1, 3)
    out_sfb = _run(ids4, embed, pos4)
    return out_sfb.transpose(2, 0, 1)


# speedup vs baseline: 1.1703x; 1.1703x over previous
import jax
import jax.numpy as jnp
from jax import lax
from jax.experimental import pallas as pl
from jax.experimental.pallas import tpu as pltpu
from jax.experimental.pallas import tpu_sc as plsc

VOCAB = 1000000
DIM = 64
B = 4096
S = 200
MAXS = 2048

NC = 2
NS = 16
NW = NC * NS
LANES = 16
BC = 256
NBCH = B // BC
NCHUNK = S * NBCH
CPW = NCHUNK // NW


def _body(ids_hbm, embed_hbm, pos_hbm, out_hbm,
          iv0, iv1, r0, r1, t0, t1, pos_v,
          si0, si1, sg0, sg1, so0, so1):
    iv = [iv0, iv1]
    rows = [r0, r1]
    tr = [t0, t1]
    si = [si0, si1]
    sg = [sg0, sg1]
    so = [so0, so1]

    wid = lax.axis_index("s") * NC + lax.axis_index("c")
    g0 = wid * CPW
    iota = lax.iota(jnp.int32, LANES)
    fhi = [(iota + LANES * fb) >> 3 for fb in range(DIM // LANES)]
    flo = [(iota + LANES * fb) & 7 for fb in range(DIM // LANES)]

    for k in range(2):
        pltpu.sync_copy(pos_hbm.at[:, k], pos_v.at[:, k])

    def schunk(t):
        g = g0 + t
        bc = pl.multiple_of((g & (NBCH - 1)) << 8, BC)
        return g >> 4, bc

    def idesc(t, p):
        g = g0 + t
        s = g >> 4
        s_hi, s_lo, b_hi = s >> 3, s & 7, g & (NBCH - 1)
        b2 = b_hi << 1
        return (
            pltpu.make_async_copy(
                ids_hbm.at[s_hi, b2, s_lo], iv[p].at[pl.ds(0, 128)], si[p]),
            pltpu.make_async_copy(
                ids_hbm.at[s_hi, b2 + 1, s_lo], iv[p].at[pl.ds(128, 128)],
                si[p]),
        )

    def gdesc(t, p):
        return (
            pltpu.make_async_copy(
                embed_hbm.at[iv[p].at[pl.ds(0, 128)]],
                rows[p].at[pl.ds(0, 128)], sg[p]),
            pltpu.make_async_copy(
                embed_hbm.at[iv[p].at[pl.ds(128, 128)]],
                rows[p].at[pl.ds(128, 128)], sg[p]),
        )

    def odesc(t, p):
        g = g0 + t
        s = g >> 4
        b2 = (g & (NBCH - 1)) << 1
        return (
            pltpu.make_async_copy(
                tr[p].at[:, 0, :, pl.ds(0, 128)],
                out_hbm.at[s, :, b2, :, :], so[p]),
            pltpu.make_async_copy(
                tr[p].at[:, 1, :, pl.ds(0, 128)],
                out_hbm.at[s, :, b2 + 1, :, :], so[p]),
        )

    for x in idesc(0, 0):
        x.start()
    for x in idesc(1, 1):
        x.start()
    for x in idesc(0, 0):
        x.wait()
    for x in gdesc(0, 0):
        x.start()

    def chunk_iter(tt, carry):
        for p in range(2):
            t = 2 * tt + p
            for x in gdesc(t, p):
                x.wait()

            @pl.when(t + 2 < CPW)
            def _():
                for x in idesc(t + 2, p):
                    x.start()

            @pl.when(t + 1 < CPW)
            def _():
                for x in idesc(t + 1, 1 - p):
                    x.wait()
                for x in gdesc(t + 1, 1 - p):
                    x.start()

            @pl.when(t >= 2)
            def _():
                for x in odesc(t - 2, p):
                    x.wait()

            s, _bc = schunk(t)
            shi_sp = jnp.full((LANES,), s >> 7, jnp.int32)
            slo_sp = jnp.full((LANES,), s & 127, jnp.int32)
            posv = [plsc.load_gather(pos_v, [fhi[fb], shi_sp, flo[fb], slo_sp])
                    for fb in range(DIM // LANES)]

            def jbody(j4, c2, p=p, posv=posv):
                for ju in range(4):
                    j = 4 * j4 + ju
                    bh_sp = jnp.full((LANES,), j >> 7, jnp.int32)
                    bl_sp = jnp.full((LANES,), j & 127, jnp.int32)
                    for fb in range(DIM // LANES):
                        v = rows[p][j, pl.ds(LANES * fb, LANES)]
                        plsc.store_scatter(
                            tr[p], [fhi[fb], bh_sp, flo[fb], bl_sp],
                            v + posv[fb])
                return c2

            lax.fori_loop(0, BC // 4, jbody, 0)
            for x in odesc(t, p):
                x.start()
        return carry

    lax.fori_loop(0, CPW // 2, chunk_iter, 0)

    for x in odesc(CPW - 2, 0):
        x.wait()
    for x in odesc(CPW - 1, 1):
        x.wait()


@jax.jit
def _run(ids4, embed, pos4):
    mesh = plsc.VectorSubcoreMesh(core_axis_name="c", subcore_axis_name="s")
    f = pl.kernel(
        _body,
        out_type=jax.ShapeDtypeStruct((S, 8, B // 128, 8, 128), jnp.float32),
        mesh=mesh,
        scratch_types=[
            pltpu.VMEM((2 * 128,), jnp.int32),
            pltpu.VMEM((2 * 128,), jnp.int32),
            pltpu.VMEM((BC, DIM), jnp.float32),
            pltpu.VMEM((BC, DIM), jnp.float32),
            pltpu.VMEM((8, 2, 8, 129), jnp.float32),
            pltpu.VMEM((8, 2, 8, 129), jnp.float32),
            pltpu.VMEM((8, 2, 8, 128), jnp.float32),
        ] + [pltpu.SemaphoreType.DMA] * 6,
        compiler_params=pltpu.CompilerParams(
            use_tc_tiling_on_sc=False, needs_layout_passes=False),
    )
    return f(ids4, embed, pos4)


def kernel(input_ids, embed, pos):
    ids4 = (input_ids.astype(jnp.int32).T
            .reshape(S // 8, 8, B // 128, 128).transpose(0, 2, 1, 3))
    pos4 = pos[0].T.reshape(8, 8, MAXS // 128, 128).transpose(0, 2, 1, 3)
    out5 = _run(ids4, embed, pos4)
    return out5.transpose(2, 4, 0, 1, 3).reshape(B, S, DIM)


# speedup vs baseline: 1.1726x; 1.0020x over previous
import jax
import jax.numpy as jnp
from jax import lax
from jax.experimental import pallas as pl
from jax.experimental.pallas import tpu as pltpu
from jax.experimental.pallas import tpu_sc as plsc

VOCAB = 1000000
DIM = 64
B = 4096
S = 200
MAXS = 2048

NC = 2
NS = 16
NW = NC * NS
LANES = 16
BC = 256
NBCH = B // BC
NCHUNK = S * NBCH
CPW = NCHUNK // NW


def _body(ids_hbm, embed_hbm, pos_hbm, out_hbm,
          iv0, iv1, r0, r1, t0, t1, pos_v,
          si0, si1, sg0, sg1, so0, so1):
    iv = [iv0, iv1]
    rows = [r0, r1]
    tr = [t0, t1]
    si = [si0, si1]
    sg = [sg0, sg1]
    so = [so0, so1]

    wid = lax.axis_index("s") * NC + lax.axis_index("c")
    g0 = wid * CPW
    iota = lax.iota(jnp.int32, LANES)
    fhi = [(iota + LANES * fb) >> 3 for fb in range(DIM // LANES)]
    flo = [(iota + LANES * fb) & 7 for fb in range(DIM // LANES)]

    for k in range(2):
        pltpu.sync_copy(pos_hbm.at[:, k], pos_v.at[:, k])

    def idesc(t, p):
        g = g0 + t
        s = g >> 4
        s_hi, s_lo, b_hi = s >> 3, s & 7, g & (NBCH - 1)
        b2 = b_hi << 1
        return (
            pltpu.make_async_copy(
                ids_hbm.at[s_hi, b2, s_lo], iv[p].at[pl.ds(0, 128)], si[p]),
            pltpu.make_async_copy(
                ids_hbm.at[s_hi, b2 + 1, s_lo], iv[p].at[pl.ds(128, 128)],
                si[p]),
        )

    def gdesc(t, p):
        return (
            pltpu.make_async_copy(
                embed_hbm.at[iv[p].at[pl.ds(0, 128)]],
                rows[p].at[pl.ds(0, 128)], sg[p]),
            pltpu.make_async_copy(
                embed_hbm.at[iv[p].at[pl.ds(128, 128)]],
                rows[p].at[pl.ds(128, 128)], sg[p]),
        )

    def odesc(t, p):
        g = g0 + t
        s = g >> 4
        b2 = (g & (NBCH - 1)) << 1
        return (
            pltpu.make_async_copy(
                tr[p].at[:, 0, :, pl.ds(0, 128)],
                out_hbm.at[s, :, b2, :, :], so[p]),
            pltpu.make_async_copy(
                tr[p].at[:, 1, :, pl.ds(0, 128)],
                out_hbm.at[s, :, b2 + 1, :, :], so[p]),
        )

    for x in idesc(0, 0):
        x.start()
    for x in idesc(1, 1):
        x.start()
    for x in idesc(0, 0):
        x.wait()
    for x in gdesc(0, 0):
        x.start()

    def chunk_iter(tt, carry):
        for p in range(2):
            t = 2 * tt + p
            for x in gdesc(t, p):
                x.wait()

            @pl.when(t + 2 < CPW)
            def _():
                for x in idesc(t + 2, p):
                    x.start()

            @pl.when(t + 1 < CPW)
            def _():
                for x in idesc(t + 1, 1 - p):
                    x.wait()
                for x in gdesc(t + 1, 1 - p):
                    x.start()

            @pl.when(t >= 2)
            def _():
                for x in odesc(t - 2, p):
                    x.wait()

            s = (g0 + t) >> 4
            shi_sp = jnp.full((LANES,), s >> 7, jnp.int32)
            slo_sp = jnp.full((LANES,), s & 127, jnp.int32)
            posv = [plsc.load_gather(pos_v, [fhi[fb], shi_sp, flo[fb], slo_sp])
                    for fb in range(DIM // LANES)]

            def jbody(j4, c2, p=p, posv=posv):
                for ju in range(4):
                    j = 4 * j4 + ju
                    bh_sp = jnp.full((LANES,), j >> 7, jnp.int32)
                    bl_sp = jnp.full((LANES,), j & 127, jnp.int32)
                    for fb in range(DIM // LANES):
                        v = rows[p][j, pl.ds(LANES * fb, LANES)]
                        plsc.store_scatter(
                            tr[p], [fhi[fb], bh_sp, flo[fb], bl_sp],
                            v + posv[fb])
                return c2

            lax.fori_loop(0, BC // 4, jbody, 0)
            for x in odesc(t, p):
                x.start()
        return carry

    lax.fori_loop(0, CPW // 2, chunk_iter, 0)

    for x in odesc(CPW - 2, 0):
        x.wait()
    for x in odesc(CPW - 1, 1):
        x.wait()


@jax.jit
def _run(ids4, embed, pos4):
    mesh = plsc.VectorSubcoreMesh(core_axis_name="c", subcore_axis_name="s")
    f = pl.kernel(
        _body,
        out_type=jax.ShapeDtypeStruct((S, 8, B // 128, 8, 128), jnp.float32),
        mesh=mesh,
        scratch_types=[
            pltpu.VMEM((2 * 128,), jnp.int32),
            pltpu.VMEM((2 * 128,), jnp.int32),
            pltpu.VMEM((BC, DIM), jnp.float32),
            pltpu.VMEM((BC, DIM), jnp.float32),
            pltpu.VMEM((8, 2, 8, 129), jnp.float32),
            pltpu.VMEM((8, 2, 8, 129), jnp.float32),
            pltpu.VMEM((8, 2, 8, 128), jnp.float32),
        ] + [pltpu.SemaphoreType.DMA] * 6,
        compiler_params=pltpu.CompilerParams(
            use_tc_tiling_on_sc=False, needs_layout_passes=False),
    )
    return f(ids4, embed, pos4)


def kernel(input_ids, embed, pos):
    ids4 = (input_ids.astype(jnp.int32).T
            .reshape(S // 8, 8, B // 128, 128).transpose(0, 2, 1, 3))
    pos4 = pos[0].T.reshape(8, 8, MAXS // 128, 128).transpose(0, 2, 1, 3)
    out5 = _run(ids4, embed, pos4)
    return out5.transpose(2, 4, 0, 1, 3).reshape(B, S, DIM)
